# baseline (speedup 1.0000x reference)
import functools

import jax
import jax.numpy as jnp
from jax.experimental import pallas as pl
from jax.experimental.pallas import tpu as pltpu

TEMP = 0.05
INV_TEMP = 1.0 / TEMP


def _ce_block_kernel(inputs_ref, targets_ref, feat_ref, out_ref,
                     m_ref, s_ref, tl_ref, *, nblocks, bk):
    i = pl.program_id(0)

    @pl.when(i == 0)
    def _init():
        m_ref[...] = jnp.full_like(m_ref, -jnp.inf)
        s_ref[...] = jnp.zeros_like(s_ref)
        tl_ref[...] = jnp.zeros_like(tl_ref)

    x = inputs_ref[...]
    f = feat_ref[...]
    z = jax.lax.dot_general(
        x, f, (((1,), (1,)), ((), ())),
        preferred_element_type=jnp.float32) * INV_TEMP

    b = x.shape[0]
    cols = i * bk + jax.lax.broadcasted_iota(jnp.int32, (b, bk), 1)
    t = targets_ref[...]
    tl_ref[...] += jnp.sum(jnp.where(cols == t, z, 0.0), axis=1,
                           keepdims=True)

    bm = jnp.max(z, axis=1, keepdims=True)
    m_old = m_ref[...]
    m_new = jnp.maximum(m_old, bm)
    s_ref[...] = (s_ref[...] * jnp.exp(m_old - m_new)
                  + jnp.sum(jnp.exp(z - m_new), axis=1, keepdims=True))
    m_ref[...] = m_new

    @pl.when(i == nblocks - 1)
    def _fin():
        nll = m_ref[...] + jnp.log(s_ref[...]) - tl_ref[...]
        out_ref[0, 0] = jnp.mean(nll)


def _pick_block(m):
    for bk in (25000, 20000, 12500, 10000, 8000, 5000, 4000, 2500, 2000,
               1250, 1000, 800, 500, 250, 200, 125, 100, 50, 25, 20, 10, 5):
        if m % bk == 0:
            return bk
    return m


@jax.jit
def kernel(inputs, targets, features):
    b, d = inputs.shape
    m, _ = features.shape
    bk = _pick_block(m)
    nblocks = m // bk
    t2d = targets.astype(jnp.int32).reshape(b, 1)

    out = pl.pallas_call(
        functools.partial(_ce_block_kernel, nblocks=nblocks, bk=bk),
        grid=(nblocks,),
        in_specs=[
            pl.BlockSpec((b, d), lambda i: (0, 0)),
            pl.BlockSpec((b, 1), lambda i: (0, 0)),
            pl.BlockSpec((bk, d), lambda i: (i, 0)),
        ],
        out_specs=pl.BlockSpec(memory_space=pltpu.SMEM),
        out_shape=jax.ShapeDtypeStruct((1, 1), jnp.float32),
        scratch_shapes=[
            pltpu.VMEM((b, 1), jnp.float32),
            pltpu.VMEM((b, 1), jnp.float32),
            pltpu.VMEM((b, 1), jnp.float32),
        ],
        compiler_params=pltpu.CompilerParams(
            dimension_semantics=("arbitrary",)),
    )(inputs, t2d, features)
    return out[0, 0]
